# Initial kernel scaffold; baseline (speedup 1.0000x reference)
import functools

import jax
import jax.numpy as jnp
from jax import lax
from jax.experimental import pallas as pl
from jax.experimental.pallas import tpu as pltpu
from jax.experimental.pallas import tpu_sc as plsc

BN_EPS = 1e-5

NC = 2
NS = 16
NW = NC * NS
CHUNK = 128


def _sc_segment_sum(x, src_c, dst_c, zrows, n_pad, k_chunks, rows_per_tile,
                    rows_per_tile_pad):
    n, d = x.shape

    mesh = plsc.VectorSubcoreMesh(core_axis_name="c", subcore_axis_name="s")

    @functools.partial(
        pl.kernel,
        out_type=jax.ShapeDtypeStruct((NC, n, d), jnp.float32),
        mesh=mesh,
        scratch_types=[
            pltpu.VMEM((k_chunks, CHUNK), jnp.int32),
            pltpu.VMEM((k_chunks, CHUNK), jnp.int32),
            pltpu.VMEM((CHUNK, d), jnp.float32),
            pltpu.VMEM_SHARED((n_pad, d), jnp.float32),
            pltpu.SemaphoreType.DMA,
        ],
    )
    def agg_kernel(x_hbm, src_hbm, dst_hbm, z_hbm, out_hbm,
                   src_v, dst_v, rows_v, agg_sh, sem):
        c = lax.axis_index("c")
        s = lax.axis_index("s")
        wid = c * NS + s
        pltpu.sync_copy(src_hbm.at[wid], src_v)
        pltpu.sync_copy(dst_hbm.at[wid], dst_v)
        pltpu.sync_copy(
            z_hbm, agg_sh.at[pl.ds(s * rows_per_tile_pad, rows_per_tile_pad)])
        plsc.subcore_barrier()

        def body(j, carry):
            pltpu.async_copy(x_hbm.at[src_v.at[j]], rows_v, sem).wait()
            pltpu.sync_copy(rows_v, agg_sh.at[dst_v.at[j]], add=True)
            return carry

        lax.fori_loop(0, k_chunks, body, 0)
        plsc.subcore_barrier()
        pltpu.sync_copy(
            agg_sh.at[pl.ds(s * rows_per_tile, rows_per_tile)],
            out_hbm.at[c, pl.ds(s * rows_per_tile, rows_per_tile)])

    return agg_kernel(x, src_c, dst_c, zrows)


def _mlp_body(x_ref, p0_ref, p1_ref, w1_ref, b1_ref, w2_ref, b2_ref,
              h2_ref, sum_ref, sq_ref):
    h = x_ref[...] + p0_ref[0] + p1_ref[0]
    h1 = jnp.dot(h, w1_ref[...], preferred_element_type=jnp.float32)
    h1 = jnp.maximum(h1 + b1_ref[...], 0.0)
    h2 = jnp.dot(h1, w2_ref[...], preferred_element_type=jnp.float32)
    h2 = h2 + b2_ref[...]
    h2_ref[...] = h2
    sum_ref[...] = jnp.sum(h2, axis=0, keepdims=True)
    sq_ref[...] = jnp.sum(h2 * h2, axis=0, keepdims=True)


def _bn_body(n_rows, h2_ref, x_ref, sum_ref, sq_ref, gamma_ref, beta_ref,
             o_ref):
    total = jnp.sum(sum_ref[...], axis=0, keepdims=True)
    total_sq = jnp.sum(sq_ref[...], axis=0, keepdims=True)
    mean = total / n_rows
    var = total_sq / n_rows - mean * mean
    scale = lax.rsqrt(var + BN_EPS) * gamma_ref[...]
    shift = beta_ref[...] - mean * scale
    bn = jnp.maximum(h2_ref[...] * scale + shift, 0.0)
    o_ref[...] = x_ref[...] + bn


def kernel(x, edge_index, W1, b1, W2, b2, gamma, beta):
    n, d = x.shape
    e = edge_index.shape[1]

    k_chunks = -(-e // (NW * CHUNK))
    e_pad = NW * CHUNK * k_chunks
    n_pad = n + (NS - n % NS if n % NS else NS)
    rows_per_tile = n // NS
    rows_per_tile_pad = n_pad // NS

    src = edge_index[0]
    dst = edge_index[1]
    pad = e_pad - e
    src_p = jnp.concatenate([src, jnp.zeros((pad,), jnp.int32)])
    dst_p = jnp.concatenate([dst, jnp.full((pad,), n, jnp.int32)])
    src_c = src_p.reshape(NW, k_chunks, CHUNK)
    dst_c = dst_p.reshape(NW, k_chunks, CHUNK)
    zrows = jnp.zeros((rows_per_tile_pad, d), jnp.float32)

    partials = _sc_segment_sum(x, src_c, dst_c, zrows, n_pad, k_chunks,
                               rows_per_tile, rows_per_tile_pad)

    blk = 1000
    grid = n // blk
    b1r = b1.reshape(1, d)
    b2r = b2.reshape(1, d)
    gammar = gamma.reshape(1, d)
    betar = beta.reshape(1, d)

    h2, sums, sqs = pl.pallas_call(
        _mlp_body,
        grid=(grid,),
        in_specs=[
            pl.BlockSpec((blk, d), lambda i: (i, 0)),
            pl.BlockSpec((1, blk, d), lambda i: (0, i, 0)),
            pl.BlockSpec((1, blk, d), lambda i: (1, i, 0)),
            pl.BlockSpec((d, d), lambda i: (0, 0)),
            pl.BlockSpec((1, d), lambda i: (0, 0)),
            pl.BlockSpec((d, d), lambda i: (0, 0)),
            pl.BlockSpec((1, d), lambda i: (0, 0)),
        ],
        out_specs=[
            pl.BlockSpec((blk, d), lambda i: (i, 0)),
            pl.BlockSpec((1, d), lambda i: (i, 0)),
            pl.BlockSpec((1, d), lambda i: (i, 0)),
        ],
        out_shape=[
            jax.ShapeDtypeStruct((n, d), jnp.float32),
            jax.ShapeDtypeStruct((grid, d), jnp.float32),
            jax.ShapeDtypeStruct((grid, d), jnp.float32),
        ],
    )(x, partials, partials, W1, b1r, W2, b2r)

    out = pl.pallas_call(
        functools.partial(_bn_body, float(n)),
        grid=(grid,),
        in_specs=[
            pl.BlockSpec((blk, d), lambda i: (i, 0)),
            pl.BlockSpec((blk, d), lambda i: (i, 0)),
            pl.BlockSpec((grid, d), lambda i: (0, 0)),
            pl.BlockSpec((grid, d), lambda i: (0, 0)),
            pl.BlockSpec((1, d), lambda i: (0, 0)),
            pl.BlockSpec((1, d), lambda i: (0, 0)),
        ],
        out_specs=pl.BlockSpec((blk, d), lambda i: (i, 0)),
        out_shape=jax.ShapeDtypeStruct((n, d), jnp.float32),
    )(h2, x, sums, sqs, gammar, betar)

    return out


# speedup vs baseline: 4.9634x; 4.9634x over previous
import functools

import jax
import jax.numpy as jnp
from jax import lax
from jax.experimental import pallas as pl
from jax.experimental.pallas import tpu as pltpu
from jax.experimental.pallas import tpu_sc as plsc

BN_EPS = 1e-5

NC = 2
NS = 16
NW = NC * NS
CHUNK = 128


def _sc_segment_sum(x, src_c, dst_c, zrows, n_pad, k_chunks, rows_per_tile):
    n, d = x.shape

    mesh = plsc.VectorSubcoreMesh(core_axis_name="c", subcore_axis_name="s")

    @functools.partial(
        pl.kernel,
        out_type=jax.ShapeDtypeStruct((NC, n_pad, d), jnp.float32),
        mesh=mesh,
        scratch_types=[
            pltpu.VMEM((k_chunks, CHUNK), jnp.int32),
            pltpu.VMEM((k_chunks, CHUNK), jnp.int32),
            pltpu.VMEM((CHUNK, d), jnp.float32),
            pltpu.VMEM_SHARED((n_pad, d), jnp.float32),
            pltpu.SemaphoreType.DMA,
        ],
    )
    def agg_kernel(x_hbm, src_hbm, dst_hbm, z_hbm, out_hbm,
                   src_v, dst_v, rows_v, agg_sh, sem):
        c = lax.axis_index("c")
        s = lax.axis_index("s")
        wid = c * NS + s
        pltpu.sync_copy(src_hbm.at[wid], src_v)
        pltpu.sync_copy(dst_hbm.at[wid], dst_v)
        pltpu.sync_copy(
            z_hbm, agg_sh.at[pl.ds(s * rows_per_tile, rows_per_tile)])
        plsc.subcore_barrier()

        def body(j, carry):
            pltpu.async_copy(x_hbm.at[src_v.at[j]], rows_v, sem).wait()
            pltpu.sync_copy(rows_v, agg_sh.at[dst_v.at[j]], add=True)
            return carry

        lax.fori_loop(0, k_chunks, body, 0)
        plsc.subcore_barrier()
        pltpu.sync_copy(
            agg_sh.at[pl.ds(s * rows_per_tile, rows_per_tile)],
            out_hbm.at[c, pl.ds(s * rows_per_tile, rows_per_tile)])

    return agg_kernel(x, src_c, dst_c, zrows)


def _mlp_body(x_ref, p0_ref, p1_ref, w1_ref, b1_ref, w2_ref, b2_ref,
              h2_ref, sum_ref, sq_ref):
    h = x_ref[...] + p0_ref[0] + p1_ref[0]
    h1 = jnp.dot(h, w1_ref[...], preferred_element_type=jnp.float32)
    h1 = jnp.maximum(h1 + b1_ref[...], 0.0)
    h2 = jnp.dot(h1, w2_ref[...], preferred_element_type=jnp.float32)
    h2 = h2 + b2_ref[...]
    h2_ref[...] = h2
    sum_ref[...] = jnp.sum(h2, axis=0, keepdims=True)[None]
    sq_ref[...] = jnp.sum(h2 * h2, axis=0, keepdims=True)[None]


def _bn_body(n_rows, h2_ref, x_ref, sum_ref, sq_ref, gamma_ref, beta_ref,
             o_ref):
    total = jnp.sum(sum_ref[...], axis=(0, 1))[None]
    total_sq = jnp.sum(sq_ref[...], axis=(0, 1))[None]
    mean = total / n_rows
    var = total_sq / n_rows - mean * mean
    scale = lax.rsqrt(var + BN_EPS) * gamma_ref[...]
    shift = beta_ref[...] - mean * scale
    bn = jnp.maximum(h2_ref[...] * scale + shift, 0.0)
    o_ref[...] = x_ref[...] + bn


def kernel(x, edge_index, W1, b1, W2, b2, gamma, beta):
    n, d = x.shape
    e = edge_index.shape[1]

    k_chunks = -(-e // (NW * CHUNK))
    e_pad = NW * CHUNK * k_chunks
    n_pad = (n // (NS * 8) + 1) * (NS * 8)
    rows_per_tile = n_pad // NS

    src = edge_index[0]
    dst = edge_index[1]
    pad = e_pad - e
    src_p = jnp.concatenate([src, jnp.zeros((pad,), jnp.int32)])
    dst_p = jnp.concatenate([dst, jnp.full((pad,), n, jnp.int32)])
    src_c = src_p.reshape(NW, k_chunks, CHUNK)
    dst_c = dst_p.reshape(NW, k_chunks, CHUNK)
    zrows = jnp.zeros((rows_per_tile, d), jnp.float32)

    partials = _sc_segment_sum(x, src_c, dst_c, zrows, n_pad, k_chunks,
                               rows_per_tile)

    blk = 1000
    grid = n // blk
    b1r = b1.reshape(1, d)
    b2r = b2.reshape(1, d)
    gammar = gamma.reshape(1, d)
    betar = beta.reshape(1, d)

    h2, sums, sqs = pl.pallas_call(
        _mlp_body,
        grid=(grid,),
        in_specs=[
            pl.BlockSpec((blk, d), lambda i: (i, 0)),
            pl.BlockSpec((1, blk, d), lambda i: (0, i, 0)),
            pl.BlockSpec((1, blk, d), lambda i: (1, i, 0)),
            pl.BlockSpec((d, d), lambda i: (0, 0)),
            pl.BlockSpec((1, d), lambda i: (0, 0)),
            pl.BlockSpec((d, d), lambda i: (0, 0)),
            pl.BlockSpec((1, d), lambda i: (0, 0)),
        ],
        out_specs=[
            pl.BlockSpec((blk, d), lambda i: (i, 0)),
            pl.BlockSpec((1, 1, d), lambda i: (i, 0, 0)),
            pl.BlockSpec((1, 1, d), lambda i: (i, 0, 0)),
        ],
        out_shape=[
            jax.ShapeDtypeStruct((n, d), jnp.float32),
            jax.ShapeDtypeStruct((grid, 1, d), jnp.float32),
            jax.ShapeDtypeStruct((grid, 1, d), jnp.float32),
        ],
    )(x, partials, partials, W1, b1r, W2, b2r)

    out = pl.pallas_call(
        functools.partial(_bn_body, float(n)),
        grid=(grid,),
        in_specs=[
            pl.BlockSpec((blk, d), lambda i: (i, 0)),
            pl.BlockSpec((blk, d), lambda i: (i, 0)),
            pl.BlockSpec((grid, 1, d), lambda i: (0, 0, 0)),
            pl.BlockSpec((grid, 1, d), lambda i: (0, 0, 0)),
            pl.BlockSpec((1, d), lambda i: (0, 0)),
            pl.BlockSpec((1, d), lambda i: (0, 0)),
        ],
        out_specs=pl.BlockSpec((blk, d), lambda i: (i, 0)),
        out_shape=jax.ShapeDtypeStruct((n, d), jnp.float32),
    )(h2, x, sums, sqs, gammar, betar)

    return out
